# baseline (speedup 1.0000x reference)
import functools

import jax
import jax.numpy as jnp
from jax import lax
from jax.experimental import pallas as pl
from jax.experimental.pallas import tpu as pltpu
from jax.experimental.pallas import tpu_sc as plsc

N = 4
VOCAB = 1000000
D = 16
TOT = 4096 * 50
NC = 2
NS = 16
NW = NC * NS
BPW = TOT // NW
CH = 128
NCH = BPW // CH
LANES = 16
SCALE = 4.0


def _body(x_hbm, w_hbm, out_hbm, idx_v, idxt_v, rows_v, sem):
  wid = lax.axis_index("s") * NC + lax.axis_index("c")
  base = wid * BPW
  pltpu.sync_copy(x_hbm.at[wid], idx_v)

  for t in range(N):

    def chunk_body(j, _, t=t):
      for k in range(CH // LANES):
        sl = pl.ds(k * LANES, LANES)
        idxt_v[sl] = idx_v[j, sl] + t * VOCAB
      pltpu.async_copy(w_hbm.at[idxt_v], rows_v, sem).wait()
      for i in range(CH):
        rows_v[i] = rows_v[i] * SCALE
      pltpu.sync_copy(rows_v, out_hbm.at[t].at[pl.ds(base + j * CH, CH)])
      return 0

    lax.fori_loop(0, NCH, chunk_body, 0)


def kernel(x, W):
  xr = x.reshape(NW, NCH, CH).astype(jnp.int32)
  wf = W.reshape(N * VOCAB, D)
  mesh = plsc.VectorSubcoreMesh(
      core_axis_name="c", subcore_axis_name="s", num_cores=NC,
      num_subcores=NS)
  call = pl.kernel(
      _body,
      out_type=jax.ShapeDtypeStruct((N, TOT, D), jnp.float32),
      mesh=mesh,
      scratch_types=[
          pltpu.VMEM((NCH, CH), jnp.int32),
          pltpu.VMEM((CH,), jnp.int32),
          pltpu.VMEM((CH, D), jnp.float32),
          pltpu.SemaphoreType.DMA,
      ],
      compiler_params=pltpu.CompilerParams(use_tc_tiling_on_sc=False),
  )
  out = call(xr, wf)
  b, l = x.shape
  return tuple(out[i].reshape(b, l, D) for i in range(N))
